# Initial kernel scaffold; baseline (speedup 1.0000x reference)
import jax
import jax.numpy as jnp
from jax.experimental import pallas as pl


def kernel(inputs, decoder_segment_ids, decoder_positions, w_ln1, w_ln2, wq, wk, wv, wo_attn, router_w, wi_gate, wi_up, wo_moe):
    raise NotImplementedError("write your pallas kernel here")


# speedup vs baseline: 1.1314x; 1.1314x over previous
import functools
import math

import jax
import jax.numpy as jnp
from jax.experimental import pallas as pl
from jax.experimental.pallas import tpu as pltpu

S, D = 2048, 1024
H, KV, HD = 16, 8, 64
E, NK, F = 8, 2, 2048
EPS = 1e-6
BM = 256
NEG = -1e9


def _qkv_kernel(x_ref, ln1_ref, wq_ref, wk_ref, wv_ref, q_ref, k_ref, v_ref):
    i = pl.program_id(0)
    x = x_ref[...]
    var = jnp.mean(jnp.square(x), axis=-1, keepdims=True)
    lnx = (x * jax.lax.rsqrt(var + EPS) * ln1_ref[...]).astype(jnp.bfloat16)
    q = jnp.dot(lnx, wq_ref[...], preferred_element_type=jnp.float32)
    k = jnp.dot(lnx, wk_ref[...], preferred_element_type=jnp.float32)
    v = jnp.dot(lnx, wv_ref[...], preferred_element_type=jnp.float32)
    pos = (i * BM
           + jax.lax.broadcasted_iota(jnp.int32, (BM, 1), 0)).astype(jnp.float32)
    half = HD // 2
    inv_freq = 1.0 / (10000.0 ** (
        jax.lax.broadcasted_iota(jnp.int32, (1, half), 1).astype(jnp.float32)
        / half))
    ang = pos * inv_freq
    cos = jnp.cos(ang)
    sin = jnp.sin(ang)

    def rope(t, nh):
        outs = []
        for h in range(nh):
            x1 = t[:, h * HD:h * HD + half]
            x2 = t[:, h * HD + half:(h + 1) * HD]
            outs.append(x1 * cos - x2 * sin)
            outs.append(x1 * sin + x2 * cos)
        return jnp.concatenate(outs, axis=1)

    q_ref[...] = rope(q, H)
    k_ref[...] = rope(k, KV)
    v_ref[...] = v


def _attn_kernel(q_ref, k_ref, v_ref, o_ref):
    i = pl.program_id(1)
    q = q_ref[0]
    k = k_ref[0]
    logits = jax.lax.dot_general(
        q, k, (((1,), (1,)), ((), ())),
        preferred_element_type=jnp.float32) * (1.0 / math.sqrt(HD))
    qpos = i * BM + jax.lax.broadcasted_iota(jnp.int32, (BM, S), 0)
    kpos = jax.lax.broadcasted_iota(jnp.int32, (BM, S), 1)
    logits = jnp.where(qpos >= kpos, logits, NEG)
    m = jnp.max(logits, axis=-1, keepdims=True)
    p = jnp.exp(logits - m)
    p = p / jnp.sum(p, axis=-1, keepdims=True)
    o_ref[0] = jnp.dot(p.astype(jnp.bfloat16), v_ref[0],
                       preferred_element_type=jnp.float32).astype(jnp.bfloat16)


def _post_kernel(ctx_ref, wo_ref, x_ref, ln2_ref, rw_ref,
                 inter_ref, hid_ref, rlog_ref):
    acc = jnp.zeros((BM, D), dtype=jnp.float32)
    for h in range(H):
        acc = acc + jnp.dot(ctx_ref[h], wo_ref[h],
                            preferred_element_type=jnp.float32)
    inter = x_ref[...] + acc
    inter_ref[...] = inter
    var = jnp.mean(jnp.square(inter), axis=-1, keepdims=True)
    hid = inter * jax.lax.rsqrt(var + EPS) * ln2_ref[...]
    hidb = hid.astype(jnp.bfloat16)
    hid_ref[...] = hidb
    rlog_ref[...] = jnp.dot(hidb, rw_ref[...],
                            preferred_element_type=jnp.float32)


def _route_kernel(rl_ref, comb_ref, lb_ref):
    rl = rl_ref[...]
    lane = jax.lax.broadcasted_iota(jnp.int32, (S, E), 1)
    m1 = jnp.max(rl, axis=-1, keepdims=True)
    idx1 = jnp.min(jnp.where(rl == m1, lane, E), axis=-1, keepdims=True)
    rl2 = jnp.where(lane == idx1, NEG, rl)
    m2 = jnp.max(rl2, axis=-1, keepdims=True)
    idx2 = jnp.min(jnp.where(rl2 == m2, lane, E), axis=-1, keepdims=True)
    w1 = 1.0 / (1.0 + jnp.exp(m2 - m1))
    w2 = 1.0 - w1
    comb_ref[...] = (jnp.where(lane == idx1, w1, 0.0)
                     + jnp.where(lane == idx2, w2, 0.0))
    probs = jnp.exp(rl - m1)
    probs = probs / jnp.sum(probs, axis=-1, keepdims=True)
    pmean = jnp.mean(probs, axis=0, keepdims=True)
    fmean = jnp.mean((lane == idx1).astype(jnp.float32)
                     + (lane == idx2).astype(jnp.float32),
                     axis=0, keepdims=True)
    lb_ref[...] = E * jnp.sum(pmean * fmean, keepdims=True).reshape(1, 1)


def _moe_kernel(hid_ref, comb_ref, inter_ref, wg_ref, wu_ref, wo_ref, o_ref):
    e = pl.program_id(0)

    @pl.when(e == 0)
    def _():
        o_ref[...] = inter_ref[...]

    lane = jax.lax.broadcasted_iota(jnp.int32, (S, E), 1)
    cw = jnp.sum(jnp.where(lane == e, comb_ref[...], 0.0), axis=-1,
                 keepdims=True)
    hid = hid_ref[...]
    FB = 256
    for f in range(F // FB):
        wg = wg_ref[0, :, f * FB:(f + 1) * FB]
        wu = wu_ref[0, :, f * FB:(f + 1) * FB]
        gate = jnp.dot(hid, wg, preferred_element_type=jnp.float32)
        up = jnp.dot(hid, wu, preferred_element_type=jnp.float32)
        hmid = (gate * jax.lax.logistic(gate) * up).astype(jnp.bfloat16)
        part = jnp.dot(hmid, wo_ref[0, f * FB:(f + 1) * FB, :],
                       preferred_element_type=jnp.float32)
        o_ref[...] += cw * part


def kernel(inputs, decoder_segment_ids, decoder_positions, w_ln1, w_ln2,
           wq, wk, wv, wo_attn, router_w, wi_gate, wi_up, wo_moe):
    x = inputs.reshape(S, D)
    wq2 = wq.reshape(D, H * HD).astype(jnp.bfloat16)
    wk2 = wk.reshape(D, KV * HD).astype(jnp.bfloat16)
    wv2 = wv.reshape(D, KV * HD).astype(jnp.bfloat16)

    q, k, v = pl.pallas_call(
        _qkv_kernel,
        grid=(S // BM,),
        in_specs=[
            pl.BlockSpec((BM, D), lambda i: (i, 0)),
            pl.BlockSpec((1, D), lambda i: (0, 0)),
            pl.BlockSpec((D, H * HD), lambda i: (0, 0)),
            pl.BlockSpec((D, KV * HD), lambda i: (0, 0)),
            pl.BlockSpec((D, KV * HD), lambda i: (0, 0)),
        ],
        out_specs=[
            pl.BlockSpec((BM, H * HD), lambda i: (i, 0)),
            pl.BlockSpec((BM, KV * HD), lambda i: (i, 0)),
            pl.BlockSpec((BM, KV * HD), lambda i: (i, 0)),
        ],
        out_shape=[
            jax.ShapeDtypeStruct((S, H * HD), jnp.float32),
            jax.ShapeDtypeStruct((S, KV * HD), jnp.float32),
            jax.ShapeDtypeStruct((S, KV * HD), jnp.float32),
        ],
    )(x, w_ln1.reshape(1, D), wq2, wk2, wv2)

    qt = q.reshape(S, H, HD).transpose(1, 0, 2).astype(jnp.bfloat16)
    kt = k.reshape(S, KV, HD).transpose(1, 0, 2).astype(jnp.bfloat16)
    vt = v.reshape(S, KV, HD).transpose(1, 0, 2).astype(jnp.bfloat16)

    ctx = pl.pallas_call(
        _attn_kernel,
        grid=(H, S // BM),
        in_specs=[
            pl.BlockSpec((1, BM, HD), lambda h, i: (h, i, 0)),
            pl.BlockSpec((1, S, HD), lambda h, i: (h // (H // KV), 0, 0)),
            pl.BlockSpec((1, S, HD), lambda h, i: (h // (H // KV), 0, 0)),
        ],
        out_specs=pl.BlockSpec((1, BM, HD), lambda h, i: (h, i, 0)),
        out_shape=jax.ShapeDtypeStruct((H, S, HD), jnp.bfloat16),
    )(qt, kt, vt)

    inter, hid, rlog = pl.pallas_call(
        _post_kernel,
        grid=(S // BM,),
        in_specs=[
            pl.BlockSpec((H, BM, HD), lambda i: (0, i, 0)),
            pl.BlockSpec((H, HD, D), lambda i: (0, 0, 0)),
            pl.BlockSpec((BM, D), lambda i: (i, 0)),
            pl.BlockSpec((1, D), lambda i: (0, 0)),
            pl.BlockSpec((D, E), lambda i: (0, 0)),
        ],
        out_specs=[
            pl.BlockSpec((BM, D), lambda i: (i, 0)),
            pl.BlockSpec((BM, D), lambda i: (i, 0)),
            pl.BlockSpec((BM, E), lambda i: (i, 0)),
        ],
        out_shape=[
            jax.ShapeDtypeStruct((S, D), jnp.float32),
            jax.ShapeDtypeStruct((S, D), jnp.bfloat16),
            jax.ShapeDtypeStruct((S, E), jnp.float32),
        ],
    )(ctx, wo_attn.astype(jnp.bfloat16), x, w_ln2.reshape(1, D),
      router_w.astype(jnp.bfloat16))

    comb, lb = pl.pallas_call(
        _route_kernel,
        grid=(1,),
        in_specs=[pl.BlockSpec((S, E), lambda i: (0, 0))],
        out_specs=[
            pl.BlockSpec((S, E), lambda i: (0, 0)),
            pl.BlockSpec((1, 1), lambda i: (0, 0)),
        ],
        out_shape=[
            jax.ShapeDtypeStruct((S, E), jnp.float32),
            jax.ShapeDtypeStruct((1, 1), jnp.float32),
        ],
    )(rlog)

    out = pl.pallas_call(
        _moe_kernel,
        grid=(E,),
        in_specs=[
            pl.BlockSpec((S, D), lambda e: (0, 0)),
            pl.BlockSpec((S, E), lambda e: (0, 0)),
            pl.BlockSpec((S, D), lambda e: (0, 0)),
            pl.BlockSpec((1, D, F), lambda e: (e, 0, 0)),
            pl.BlockSpec((1, D, F), lambda e: (e, 0, 0)),
            pl.BlockSpec((1, F, D), lambda e: (e, 0, 0)),
        ],
        out_specs=pl.BlockSpec((S, D), lambda e: (0, 0)),
        out_shape=jax.ShapeDtypeStruct((S, D), jnp.float32),
    )(hid, comb, inter,
      wi_gate.astype(jnp.bfloat16), wi_up.astype(jnp.bfloat16),
      wo_moe.astype(jnp.bfloat16))

    return out.reshape(1, S, D), lb.reshape(())
